# Initial kernel scaffold; baseline (speedup 1.0000x reference)
import functools

import jax
import jax.numpy as jnp
from jax import lax
from jax.experimental import pallas as pl
from jax.experimental.pallas import tpu as pltpu
from jax.experimental.pallas import tpu_sc as plsc

N = 10000
E = 320000
D = 128
DE = 16
L = 3
ALPHA = 0.5

NC = 2
NS = 16
NW = NC * NS
C = 128
EW = (E + NW * C - 1) // (NW * C) * C
NCHUNK = EW // C
E_PAD = EW * NW
N_TAB = 10016
ROWS_PER_TILE = N_TAB // NS


def _proj_body(x_ref, ws_ref, wd_ref, os_ref, od_ref):
    x = x_ref[...]
    os_ref[...] = jnp.dot(x, ws_ref[...], preferred_element_type=jnp.float32)
    od_ref[...] = jnp.dot(x, wd_ref[...], preferred_element_type=jnp.float32)


_proj_call = pl.pallas_call(
    _proj_body,
    out_shape=(
        jax.ShapeDtypeStruct((N_TAB, DE), jnp.float32),
        jax.ShapeDtypeStruct((N_TAB, DE), jnp.float32),
    ),
)


def _elin_body(e_ref, w_ref, b_ref, o_ref):
    o_ref[...] = (
        jnp.dot(e_ref[...], w_ref[...], preferred_element_type=jnp.float32)
        + b_ref[...]
    )


_ELIN_BLK = 4096

_elin_call = pl.pallas_call(
    _elin_body,
    grid=(E_PAD // 8 // _ELIN_BLK,),
    in_specs=[
        pl.BlockSpec((_ELIN_BLK, 128), lambda i: (i, 0)),
        pl.BlockSpec((128, 128), lambda i: (0, 0)),
        pl.BlockSpec((1, 128), lambda i: (0, 0)),
    ],
    out_specs=pl.BlockSpec((_ELIN_BLK, 128), lambda i: (i, 0)),
    out_shape=jax.ShapeDtypeStruct((E_PAD // 8, 128), jnp.float32),
)


def _node_body(x_ref, ap_ref, wx_ref, wa_ref, b_ref, xres_ref, o_ref, *, with_res):
    agg = ap_ref[0] + ap_ref[1]
    d = jnp.maximum(
        jnp.dot(x_ref[...], wx_ref[...], preferred_element_type=jnp.float32)
        + jnp.dot(agg, wa_ref[...], preferred_element_type=jnp.float32)
        + b_ref[...],
        0.0,
    )
    if with_res:
        d = ALPHA * xres_ref[...] + (1.0 - ALPHA) * d
    rows = lax.broadcasted_iota(jnp.int32, (N_TAB, 1), 0)
    o_ref[...] = jnp.where(rows < N, d, 0.0)


_node_call_first = pl.pallas_call(
    functools.partial(_node_body, with_res=False),
    out_shape=jax.ShapeDtypeStruct((N_TAB, D), jnp.float32),
)

_node_call_res = pl.pallas_call(
    functools.partial(_node_body, with_res=True),
    out_shape=jax.ShapeDtypeStruct((N_TAB, D), jnp.float32),
)


_sc_mesh = plsc.VectorSubcoreMesh(
    core_axis_name="c", subcore_axis_name="s", num_cores=NC, num_subcores=NS
)


@functools.partial(
    pl.kernel,
    out_type=(
        jax.ShapeDtypeStruct((E_PAD, DE), jnp.float32),
        jax.ShapeDtypeStruct((NC, N_TAB, DE), jnp.float32),
    ),
    mesh=_sc_mesh,
    scratch_types=[
        pltpu.VMEM((NCHUNK, C), jnp.int32),
        pltpu.VMEM((NCHUNK, C), jnp.int32),
        pltpu.VMEM((C, DE), jnp.float32),
        pltpu.VMEM((C, DE), jnp.float32),
        pltpu.VMEM((C, DE), jnp.float32),
        pltpu.VMEM((C, DE), jnp.float32),
        pltpu.VMEM_SHARED((N_TAB, DE), jnp.float32),
        pltpu.SemaphoreType.DMA,
        pltpu.SemaphoreType.DMA,
    ],
)
def _sc_edge_kernel(ps_hbm, pd_hbm, elin_hbm, src_hbm, dst_hbm, zeros_hbm,
                    enew_hbm, agg_hbm,
                    src_v, dst_v, ps_v, pd_v, el_v, en_v, agg_s, sem1, sem2):
    c = lax.axis_index("c")
    s = lax.axis_index("s")
    wid = s * NC + c

    pltpu.sync_copy(
        zeros_hbm.at[pl.ds(s * ROWS_PER_TILE, ROWS_PER_TILE)],
        agg_s.at[pl.ds(s * ROWS_PER_TILE, ROWS_PER_TILE)],
    )
    pltpu.sync_copy(src_hbm.at[wid], src_v)
    pltpu.sync_copy(dst_hbm.at[wid], dst_v)
    plsc.subcore_barrier()

    ew_base = wid * EW

    def chunk_body(j, carry):
        base = ew_base + j * C
        cp1 = pltpu.async_copy(ps_hbm.at[src_v.at[j]], ps_v, sem1)
        cp2 = pltpu.async_copy(pd_hbm.at[dst_v.at[j]], pd_v, sem2)
        pltpu.sync_copy(elin_hbm.at[pl.ds(base, C)], el_v)
        cp1.wait()
        cp2.wait()

        def row_body(i, carry2):
            v = ps_v[i] + pd_v[i] + el_v[i]
            en_v[i] = jnp.maximum(v, 0.0)
            return carry2

        lax.fori_loop(0, C, row_body, 0)
        pltpu.sync_copy(en_v, enew_hbm.at[pl.ds(base, C)])
        pltpu.sync_copy(en_v, agg_s.at[dst_v.at[j]], add=True)
        return carry

    lax.fori_loop(0, NCHUNK, chunk_body, 0)

    plsc.subcore_barrier()
    pltpu.sync_copy(
        agg_s.at[pl.ds(s * ROWS_PER_TILE, ROWS_PER_TILE)],
        agg_hbm.at[c, pl.ds(s * ROWS_PER_TILE, ROWS_PER_TILE)],
    )


def kernel(x, edge_index, edge_attr, We, be, Wn, bn):
    src = edge_index[0]
    dst = edge_index[1]

    x_pad = jnp.zeros((N_TAB, D), jnp.float32).at[:N].set(x)
    src_r = jnp.zeros((E_PAD,), jnp.int32).at[:E].set(src).reshape(NW, NCHUNK, C)
    dst_r = jnp.full((E_PAD,), N, jnp.int32).at[:E].set(dst).reshape(NW, NCHUNK, C)
    e_pad = jnp.zeros((E_PAD, DE), jnp.float32).at[:E].set(edge_attr)
    zeros_tab = jnp.zeros((N_TAB, DE), jnp.float32)
    eye8 = jnp.eye(8, dtype=jnp.float32)

    x_cur = x_pad
    e_cur = e_pad
    e_outs = [edge_attr]
    x_res = None
    for i in range(L):
        ws = We[i, :D]
        wd = We[i, D:2 * D]
        wee = We[i, 2 * D:]
        ps, pd = _proj_call(x_cur, ws, wd)
        wbd = jnp.kron(eye8, wee)
        brow = jnp.tile(be[i], 8)[None, :]
        elin = _elin_call(e_cur.reshape(E_PAD // 8, 128), wbd, brow)
        elin = elin.reshape(E_PAD, DE)
        e_new, agg_p = _sc_edge_kernel(ps, pd, elin, src_r, dst_r, zeros_tab)
        wx = Wn[i, :D]
        wa = Wn[i, D:]
        bnr = bn[i][None, :]
        if i == 0:
            x_cur = _node_call_first(x_cur, agg_p, wx, wa, bnr, x_cur)
            x_res = x_cur
        else:
            x_cur = _node_call_res(x_cur, agg_p, wx, wa, bnr, x_res)
        e_cur = e_new
        e_outs.append(e_new[:E])

    return x_cur[:N], e_cur[:E], jnp.stack(e_outs, axis=0)


# speedup vs baseline: 4.1498x; 4.1498x over previous
import functools

import jax
import jax.numpy as jnp
from jax import lax
from jax.experimental import pallas as pl
from jax.experimental.pallas import tpu as pltpu
from jax.experimental.pallas import tpu_sc as plsc

N = 10000
E = 320000
D = 128
DE = 16
L = 3
ALPHA = 0.5

NC = 2
NS = 16
NW = NC * NS
C = 128
NCHUNK = 80
EW = NCHUNK * C
E_PAD = EW * NW
N_TAB = 10112
ROWS_PER_TILE = N_TAB // NS


def _proj_body(x_ref, ws_ref, wd_ref, os_ref, od_ref):
    x = x_ref[...]
    os_ref[...] = jnp.dot(x, ws_ref[...], preferred_element_type=jnp.float32)
    od_ref[...] = jnp.dot(x, wd_ref[...], preferred_element_type=jnp.float32)


_proj_call = pl.pallas_call(
    _proj_body,
    out_shape=(
        jax.ShapeDtypeStruct((N_TAB, DE), jnp.float32),
        jax.ShapeDtypeStruct((N_TAB, DE), jnp.float32),
    ),
)


def _elin_body(e_ref, w_ref, b_ref, o_ref):
    o_ref[...] = (
        jnp.dot(e_ref[...], w_ref[...], preferred_element_type=jnp.float32)
        + b_ref[...]
    )


_ELIN_BLK = 4096

_elin_call = pl.pallas_call(
    _elin_body,
    grid=(E_PAD // 8 // _ELIN_BLK,),
    in_specs=[
        pl.BlockSpec((_ELIN_BLK, 128), lambda i: (i, 0)),
        pl.BlockSpec((128, 128), lambda i: (0, 0)),
        pl.BlockSpec((1, 128), lambda i: (0, 0)),
    ],
    out_specs=pl.BlockSpec((_ELIN_BLK, 128), lambda i: (i, 0)),
    out_shape=jax.ShapeDtypeStruct((E_PAD // 8, 128), jnp.float32),
)


def _node_body(x_ref, ap_ref, wx_ref, wa_ref, b_ref, xres_ref, o_ref, *, with_res):
    agg = ap_ref[0] + ap_ref[1]
    d = jnp.maximum(
        jnp.dot(x_ref[...], wx_ref[...], preferred_element_type=jnp.float32)
        + jnp.dot(agg, wa_ref[...], preferred_element_type=jnp.float32)
        + b_ref[...],
        0.0,
    )
    if with_res:
        d = ALPHA * xres_ref[...] + (1.0 - ALPHA) * d
    rows = lax.broadcasted_iota(jnp.int32, (N_TAB, 1), 0)
    o_ref[...] = jnp.where(rows < N, d, 0.0)


_node_call_first = pl.pallas_call(
    functools.partial(_node_body, with_res=False),
    out_shape=jax.ShapeDtypeStruct((N_TAB, D), jnp.float32),
)

_node_call_res = pl.pallas_call(
    functools.partial(_node_body, with_res=True),
    out_shape=jax.ShapeDtypeStruct((N_TAB, D), jnp.float32),
)


_sc_mesh = plsc.VectorSubcoreMesh(
    core_axis_name="c", subcore_axis_name="s", num_cores=NC, num_subcores=NS
)


@functools.partial(
    pl.kernel,
    out_type=(
        jax.ShapeDtypeStruct((E_PAD, DE), jnp.float32),
        jax.ShapeDtypeStruct((NC, N_TAB, DE), jnp.float32),
    ),
    mesh=_sc_mesh,
    compiler_params=pltpu.CompilerParams(use_tc_tiling_on_sc=False),
    scratch_types=[
        pltpu.VMEM((C,), jnp.int32),
        pltpu.VMEM((C,), jnp.int32),
        pltpu.VMEM((C, DE), jnp.float32),
        pltpu.VMEM((C, DE), jnp.float32),
        pltpu.VMEM((C, DE), jnp.float32),
        pltpu.VMEM((C, DE), jnp.float32),
        pltpu.VMEM_SHARED((N_TAB, DE), jnp.float32),
        pltpu.SemaphoreType.DMA,
        pltpu.SemaphoreType.DMA,
    ],
)
def _sc_edge_kernel(ps_hbm, pd_hbm, elin_hbm, src_hbm, dst_hbm, zeros_hbm,
                    enew_hbm, agg_hbm,
                    src_v, dst_v, ps_v, pd_v, el_v, en_v, agg_s, sem1, sem2):
    c = lax.axis_index("c")
    s = lax.axis_index("s")
    wid = s * NC + c

    pltpu.sync_copy(
        zeros_hbm.at[pl.ds(s * ROWS_PER_TILE, ROWS_PER_TILE)],
        agg_s.at[pl.ds(s * ROWS_PER_TILE, ROWS_PER_TILE)],
    )
    plsc.subcore_barrier()

    ew_base = wid * EW

    def chunk_body(j, carry):
        base = ew_base + j * C
        pltpu.sync_copy(src_hbm.at[wid, j], src_v)
        pltpu.sync_copy(dst_hbm.at[wid, j], dst_v)
        cp1 = pltpu.async_copy(ps_hbm.at[src_v], ps_v, sem1)
        cp2 = pltpu.async_copy(pd_hbm.at[dst_v], pd_v, sem2)
        pltpu.sync_copy(elin_hbm.at[pl.ds(base, C)], el_v)
        cp1.wait()
        cp2.wait()

        def row_body(i, carry2):
            v = ps_v[i] + pd_v[i] + el_v[i]
            en_v[i] = jnp.maximum(v, 0.0)
            return carry2

        lax.fori_loop(0, C, row_body, 0)
        pltpu.sync_copy(en_v, enew_hbm.at[pl.ds(base, C)])
        pltpu.sync_copy(en_v, agg_s.at[dst_v], add=True)
        return carry

    lax.fori_loop(0, NCHUNK, chunk_body, 0)

    plsc.subcore_barrier()
    pltpu.sync_copy(
        agg_s.at[pl.ds(s * ROWS_PER_TILE, ROWS_PER_TILE)],
        agg_hbm.at[c, pl.ds(s * ROWS_PER_TILE, ROWS_PER_TILE)],
    )


def kernel(x, edge_index, edge_attr, We, be, Wn, bn):
    src = edge_index[0]
    dst = edge_index[1]

    x_pad = jnp.zeros((N_TAB, D), jnp.float32).at[:N].set(x)
    src_r = jnp.zeros((E_PAD,), jnp.int32).at[:E].set(src).reshape(NW, NCHUNK, C)
    dst_r = jnp.full((E_PAD,), N, jnp.int32).at[:E].set(dst).reshape(NW, NCHUNK, C)
    e_pad = jnp.zeros((E_PAD, DE), jnp.float32).at[:E].set(edge_attr)
    zeros_tab = jnp.zeros((N_TAB, DE), jnp.float32)
    eye8 = jnp.eye(8, dtype=jnp.float32)

    x_cur = x_pad
    e_cur = e_pad
    e_outs = [edge_attr]
    x_res = None
    for i in range(L):
        ws = We[i, :D]
        wd = We[i, D:2 * D]
        wee = We[i, 2 * D:]
        ps, pd = _proj_call(x_cur, ws, wd)
        wbd = jnp.kron(eye8, wee)
        brow = jnp.tile(be[i], 8)[None, :]
        elin = _elin_call(e_cur.reshape(E_PAD // 8, 128), wbd, brow)
        elin = elin.reshape(E_PAD, DE)
        e_new, agg_p = _sc_edge_kernel(ps, pd, elin, src_r, dst_r, zeros_tab)
        wx = Wn[i, :D]
        wa = Wn[i, D:]
        bnr = bn[i][None, :]
        if i == 0:
            x_cur = _node_call_first(x_cur, agg_p, wx, wa, bnr, x_cur)
            x_res = x_cur
        else:
            x_cur = _node_call_res(x_cur, agg_p, wx, wa, bnr, x_res)
        e_cur = e_new
        e_outs.append(e_new[:E])

    return x_cur[:N], e_cur[:E], jnp.stack(e_outs, axis=0)


# speedup vs baseline: 4.2116x; 1.0149x over previous
import functools

import jax
import jax.numpy as jnp
from jax import lax
from jax.experimental import pallas as pl
from jax.experimental.pallas import tpu as pltpu
from jax.experimental.pallas import tpu_sc as plsc

N = 10000
E = 320000
D = 128
DE = 16
L = 3
ALPHA = 0.5

NC = 2
NS = 16
NW = NC * NS
C = 128
NCHUNK = 80
EW = NCHUNK * C
E_PAD = EW * NW
N_TAB = 10112
ROWS_PER_TILE = N_TAB // NS


NR = N_TAB // 8
N_ROWS_REAL = N // 8


def _proj_body(x_ref, ws_ref, wd_ref, os_ref, od_ref):
    x = x_ref[...]
    os_ref[...] = jnp.dot(x, ws_ref[...], preferred_element_type=jnp.float32)
    od_ref[...] = jnp.dot(x, wd_ref[...], preferred_element_type=jnp.float32)


_proj_call = pl.pallas_call(
    _proj_body,
    out_shape=(
        jax.ShapeDtypeStruct((NR, 128), jnp.float32),
        jax.ShapeDtypeStruct((NR, 128), jnp.float32),
    ),
)


def _elin_body(e_ref, w_ref, b_ref, o_ref):
    o_ref[...] = (
        jnp.dot(e_ref[...], w_ref[...], preferred_element_type=jnp.float32)
        + b_ref[...]
    )


_ELIN_BLK = 4096

_elin_call = pl.pallas_call(
    _elin_body,
    grid=(E_PAD // 8 // _ELIN_BLK,),
    in_specs=[
        pl.BlockSpec((_ELIN_BLK, 128), lambda i: (i, 0)),
        pl.BlockSpec((128, 128), lambda i: (0, 0)),
        pl.BlockSpec((1, 128), lambda i: (0, 0)),
    ],
    out_specs=pl.BlockSpec((_ELIN_BLK, 128), lambda i: (i, 0)),
    out_shape=jax.ShapeDtypeStruct((E_PAD // 8, 128), jnp.float32),
)


def _node_body(x_ref, ap_ref, wx_ref, wa_ref, b_ref, xres_ref, o_ref, *, with_res):
    agg = ap_ref[0] + ap_ref[1]
    d = jnp.maximum(
        jnp.dot(x_ref[...], wx_ref[...], preferred_element_type=jnp.float32)
        + jnp.dot(agg, wa_ref[...], preferred_element_type=jnp.float32)
        + b_ref[...],
        0.0,
    )
    if with_res:
        d = ALPHA * xres_ref[...] + (1.0 - ALPHA) * d
    rows = lax.broadcasted_iota(jnp.int32, (NR, 1), 0)
    o_ref[...] = jnp.where(rows < N_ROWS_REAL, d, 0.0)


_node_call_first = pl.pallas_call(
    functools.partial(_node_body, with_res=False),
    out_shape=jax.ShapeDtypeStruct((NR, 8 * D), jnp.float32),
)

_node_call_res = pl.pallas_call(
    functools.partial(_node_body, with_res=True),
    out_shape=jax.ShapeDtypeStruct((NR, 8 * D), jnp.float32),
)


_sc_mesh = plsc.VectorSubcoreMesh(
    core_axis_name="c", subcore_axis_name="s", num_cores=NC, num_subcores=NS
)


@functools.partial(
    pl.kernel,
    out_type=(
        jax.ShapeDtypeStruct((E_PAD, DE), jnp.float32),
        jax.ShapeDtypeStruct((NC, N_TAB, DE), jnp.float32),
    ),
    mesh=_sc_mesh,
    compiler_params=pltpu.CompilerParams(use_tc_tiling_on_sc=False),
    scratch_types=[
        pltpu.VMEM((C,), jnp.int32),
        pltpu.VMEM((C,), jnp.int32),
        pltpu.VMEM((C, DE), jnp.float32),
        pltpu.VMEM((C, DE), jnp.float32),
        pltpu.VMEM((C, DE), jnp.float32),
        pltpu.VMEM((C, DE), jnp.float32),
        pltpu.VMEM_SHARED((N_TAB, DE), jnp.float32),
        pltpu.SemaphoreType.DMA,
        pltpu.SemaphoreType.DMA,
    ],
)
def _sc_edge_kernel(ps_hbm, pd_hbm, elin_hbm, src_hbm, dst_hbm, zeros_hbm,
                    enew_hbm, agg_hbm,
                    src_v, dst_v, ps_v, pd_v, el_v, en_v, agg_s, sem1, sem2):
    c = lax.axis_index("c")
    s = lax.axis_index("s")
    wid = s * NC + c

    pltpu.sync_copy(
        zeros_hbm.at[pl.ds(s * ROWS_PER_TILE, ROWS_PER_TILE)],
        agg_s.at[pl.ds(s * ROWS_PER_TILE, ROWS_PER_TILE)],
    )
    plsc.subcore_barrier()

    ew_base = wid * EW

    def chunk_body(j, carry):
        base = ew_base + j * C
        pltpu.sync_copy(src_hbm.at[wid, j], src_v)
        pltpu.sync_copy(dst_hbm.at[wid, j], dst_v)
        cp1 = pltpu.async_copy(ps_hbm.at[src_v], ps_v, sem1)
        cp2 = pltpu.async_copy(pd_hbm.at[dst_v], pd_v, sem2)
        pltpu.sync_copy(elin_hbm.at[pl.ds(base, C)], el_v)
        cp1.wait()
        cp2.wait()

        def row_body(i, carry2):
            v = ps_v[i] + pd_v[i] + el_v[i]
            en_v[i] = jnp.maximum(v, 0.0)
            return carry2

        lax.fori_loop(0, C, row_body, 0)
        pltpu.sync_copy(en_v, enew_hbm.at[pl.ds(base, C)])
        pltpu.sync_copy(en_v, agg_s.at[dst_v], add=True)
        return carry

    lax.fori_loop(0, NCHUNK, chunk_body, 0)

    plsc.subcore_barrier()
    pltpu.sync_copy(
        agg_s.at[pl.ds(s * ROWS_PER_TILE, ROWS_PER_TILE)],
        agg_hbm.at[c, pl.ds(s * ROWS_PER_TILE, ROWS_PER_TILE)],
    )


def kernel(x, edge_index, edge_attr, We, be, Wn, bn):
    src = edge_index[0]
    dst = edge_index[1]

    x_r = jnp.zeros((N_TAB, D), jnp.float32).at[:N].set(x).reshape(NR, 8 * D)
    src_r = jnp.zeros((E_PAD,), jnp.int32).at[:E].set(src).reshape(NW, NCHUNK, C)
    dst_r = jnp.full((E_PAD,), N, jnp.int32).at[:E].set(dst).reshape(NW, NCHUNK, C)
    e_r = (jnp.zeros((E_PAD, DE), jnp.float32).at[:E].set(edge_attr)
           .reshape(E_PAD // 8, 128))
    zeros_tab = jnp.zeros((N_TAB, DE), jnp.float32)
    eye8 = jnp.eye(8, dtype=jnp.float32)

    e_outs = [edge_attr]
    x_res = None
    for i in range(L):
        bds = jnp.kron(eye8, We[i, :D])
        bdd = jnp.kron(eye8, We[i, D:2 * D])
        ps_r, pd_r = _proj_call(x_r, bds, bdd)
        bde = jnp.kron(eye8, We[i, 2 * D:])
        brow = jnp.tile(be[i], 8)[None, :]
        elin_r = _elin_call(e_r, bde, brow)
        e_new, agg_p = _sc_edge_kernel(
            ps_r.reshape(N_TAB, DE), pd_r.reshape(N_TAB, DE),
            elin_r.reshape(E_PAD, DE), src_r, dst_r, zeros_tab)
        agg_r = agg_p.reshape(NC, NR, 128)
        bdx = jnp.kron(eye8, Wn[i, :D])
        bda = jnp.kron(eye8, Wn[i, D:])
        b_r = jnp.tile(bn[i], 8)[None, :]
        if i == 0:
            x_r = _node_call_first(x_r, agg_r, bdx, bda, b_r, x_r)
            x_res = x_r
        else:
            x_r = _node_call_res(x_r, agg_r, bdx, bda, b_r, x_res)
        e_r = e_new.reshape(E_PAD // 8, 128)
        e_outs.append(e_new[:E])

    x_out = x_r.reshape(N_TAB, D)[:N]
    return x_out, e_outs[-1], jnp.stack(e_outs, axis=0)


# speedup vs baseline: 4.9540x; 1.1763x over previous
import functools

import jax
import jax.numpy as jnp
from jax import lax
from jax.experimental import pallas as pl
from jax.experimental.pallas import tpu as pltpu
from jax.experimental.pallas import tpu_sc as plsc

N = 10000
E = 320000
D = 128
DE = 16
L = 3
ALPHA = 0.5

NC = 2
NS = 16
NW = NC * NS
C = 128
NCHUNK = 80
EW = NCHUNK * C
E_PAD = EW * NW
N_TAB = 10112
ROWS_PER_TILE = N_TAB // NS


NR = N_TAB // 8
N_ROWS_REAL = N // 8


def _proj_body(x_ref, ws_ref, wd_ref, os_ref, od_ref):
    x = x_ref[...]
    os_ref[...] = jnp.dot(x, ws_ref[...], preferred_element_type=jnp.float32)
    od_ref[...] = jnp.dot(x, wd_ref[...], preferred_element_type=jnp.float32)


_proj_call = pl.pallas_call(
    _proj_body,
    out_shape=(
        jax.ShapeDtypeStruct((NR, 128), jnp.float32),
        jax.ShapeDtypeStruct((NR, 128), jnp.float32),
    ),
)


def _elin_body(e_ref, w_ref, b_ref, o_ref):
    o_ref[...] = (
        jnp.dot(e_ref[...], w_ref[...], preferred_element_type=jnp.float32)
        + b_ref[...]
    )


_ELIN_BLK = 4096

_elin_call = pl.pallas_call(
    _elin_body,
    grid=(E_PAD // 8 // _ELIN_BLK,),
    in_specs=[
        pl.BlockSpec((_ELIN_BLK, 128), lambda i: (i, 0)),
        pl.BlockSpec((128, 128), lambda i: (0, 0)),
        pl.BlockSpec((1, 128), lambda i: (0, 0)),
    ],
    out_specs=pl.BlockSpec((_ELIN_BLK, 128), lambda i: (i, 0)),
    out_shape=jax.ShapeDtypeStruct((E_PAD // 8, 128), jnp.float32),
)


def _node_body(x_ref, ap_ref, wx_ref, wa_ref, b_ref, xres_ref, o_ref, *, with_res):
    agg = ap_ref[0] + ap_ref[1]
    d = jnp.maximum(
        jnp.dot(x_ref[...], wx_ref[...], preferred_element_type=jnp.float32)
        + jnp.dot(agg, wa_ref[...], preferred_element_type=jnp.float32)
        + b_ref[...],
        0.0,
    )
    if with_res:
        d = ALPHA * xres_ref[...] + (1.0 - ALPHA) * d
    rows = lax.broadcasted_iota(jnp.int32, (NR, 1), 0)
    o_ref[...] = jnp.where(rows < N_ROWS_REAL, d, 0.0)


_node_call_first = pl.pallas_call(
    functools.partial(_node_body, with_res=False),
    out_shape=jax.ShapeDtypeStruct((NR, 8 * D), jnp.float32),
)

_node_call_res = pl.pallas_call(
    functools.partial(_node_body, with_res=True),
    out_shape=jax.ShapeDtypeStruct((NR, 8 * D), jnp.float32),
)


_sc_mesh = plsc.VectorSubcoreMesh(
    core_axis_name="c", subcore_axis_name="s", num_cores=NC, num_subcores=NS
)


SUB = 4
R = SUB * C
NSUP = NCHUNK // SUB


@functools.partial(
    pl.kernel,
    out_type=(
        jax.ShapeDtypeStruct((E_PAD, DE), jnp.float32),
        jax.ShapeDtypeStruct((NC, N_TAB, DE), jnp.float32),
    ),
    mesh=_sc_mesh,
    compiler_params=pltpu.CompilerParams(use_tc_tiling_on_sc=False),
    scratch_types=[
        pltpu.VMEM((NCHUNK, C), jnp.int32),
        pltpu.VMEM((NCHUNK, C), jnp.int32),
        pltpu.VMEM((2, R, DE), jnp.float32),
        pltpu.VMEM((2, R, DE), jnp.float32),
        pltpu.VMEM((2, R, DE), jnp.float32),
        pltpu.VMEM((2, R, DE), jnp.float32),
        pltpu.VMEM_SHARED((N_TAB, DE), jnp.float32),
        pltpu.SemaphoreType.DMA,
        pltpu.SemaphoreType.DMA,
        pltpu.SemaphoreType.DMA,
        pltpu.SemaphoreType.DMA,
    ],
)
def _sc_edge_kernel(ps_hbm, pd_hbm, elin_hbm, src_hbm, dst_hbm, zeros_hbm,
                    enew_hbm, agg_hbm,
                    src_v, dst_v, ps_v, pd_v, el_v, en_v, agg_s,
                    sl0, sl1, ss0, ss1):
    c = lax.axis_index("c")
    s = lax.axis_index("s")
    wid = s * NC + c
    sem_l = (sl0, sl1)
    sem_s = (ss0, ss1)

    pltpu.sync_copy(
        zeros_hbm.at[pl.ds(s * ROWS_PER_TILE, ROWS_PER_TILE)],
        agg_s.at[pl.ds(s * ROWS_PER_TILE, ROWS_PER_TILE)],
    )
    pltpu.sync_copy(src_hbm.at[wid], src_v)
    pltpu.sync_copy(dst_hbm.at[wid], dst_v)
    plsc.subcore_barrier()

    ew_base = wid * EW

    def make_loads(t, p):
        ds_ = []
        for k in range(SUB):
            j = SUB * t + k
            ds_.append(pltpu.make_async_copy(
                ps_hbm.at[src_v.at[j]], ps_v.at[p, pl.ds(k * C, C)], sem_l[p]))
            ds_.append(pltpu.make_async_copy(
                pd_hbm.at[dst_v.at[j]], pd_v.at[p, pl.ds(k * C, C)], sem_l[p]))
        ds_.append(pltpu.make_async_copy(
            elin_hbm.at[pl.ds(ew_base + t * R, R)], el_v.at[p], sem_l[p]))
        return ds_

    def make_stores(t, p):
        return [pltpu.make_async_copy(
            en_v.at[p], enew_hbm.at[pl.ds(ew_base + t * R, R)], sem_s[p])]

    def start_all(ds_):
        for d in ds_:
            d.start()

    def start_stores(t, p):
        for d in make_stores(t, p):
            d.start()
        for k in range(SUB):
            j = SUB * t + k
            pltpu.sync_copy(en_v.at[p, pl.ds(k * C, C)],
                            agg_s.at[dst_v.at[j]], add=True)

    def wait_all(ds_):
        for d in ds_:
            d.wait()

    def compute(p):
        psb, pdb, elb, enb = ps_v.at[p], pd_v.at[p], el_v.at[p], en_v.at[p]

        def row_body(i, carry):
            base = i * 4
            for u in range(4):
                r = base + u
                enb[r] = jnp.maximum(psb[r] + pdb[r] + elb[r], 0.0)
            return carry

        lax.fori_loop(0, R // 4, row_body, 0)

    NQ = NSUP // 2

    start_all(make_loads(0, 0))
    start_all(make_loads(1, 1))

    def outer_body(q, carry):
        for p in range(2):
            t = 2 * q + p
            wait_all(make_loads(t, p))

            @pl.when(q > 0)
            def _():
                wait_all(make_stores(t - 2, p))

            compute(p)
            start_stores(t, p)

            @pl.when(q < NQ - 1)
            def _():
                start_all(make_loads(t + 2, p))
        return carry

    lax.fori_loop(0, NQ, outer_body, 0)
    wait_all(make_stores(NSUP - 2, 0))
    wait_all(make_stores(NSUP - 1, 1))

    plsc.subcore_barrier()
    pltpu.sync_copy(
        agg_s.at[pl.ds(s * ROWS_PER_TILE, ROWS_PER_TILE)],
        agg_hbm.at[c, pl.ds(s * ROWS_PER_TILE, ROWS_PER_TILE)],
    )


def kernel(x, edge_index, edge_attr, We, be, Wn, bn):
    src = edge_index[0]
    dst = edge_index[1]

    x_r = jnp.zeros((N_TAB, D), jnp.float32).at[:N].set(x).reshape(NR, 8 * D)
    src_r = jnp.zeros((E_PAD,), jnp.int32).at[:E].set(src).reshape(NW, NCHUNK, C)
    dst_r = jnp.full((E_PAD,), N, jnp.int32).at[:E].set(dst).reshape(NW, NCHUNK, C)
    e_r = (jnp.zeros((E_PAD, DE), jnp.float32).at[:E].set(edge_attr)
           .reshape(E_PAD // 8, 128))
    zeros_tab = jnp.zeros((N_TAB, DE), jnp.float32)
    eye8 = jnp.eye(8, dtype=jnp.float32)

    e_outs = [edge_attr]
    x_res = None
    for i in range(L):
        bds = jnp.kron(eye8, We[i, :D])
        bdd = jnp.kron(eye8, We[i, D:2 * D])
        ps_r, pd_r = _proj_call(x_r, bds, bdd)
        bde = jnp.kron(eye8, We[i, 2 * D:])
        brow = jnp.tile(be[i], 8)[None, :]
        elin_r = _elin_call(e_r, bde, brow)
        e_new, agg_p = _sc_edge_kernel(
            ps_r.reshape(N_TAB, DE), pd_r.reshape(N_TAB, DE),
            elin_r.reshape(E_PAD, DE), src_r, dst_r, zeros_tab)
        agg_r = agg_p.reshape(NC, NR, 128)
        bdx = jnp.kron(eye8, Wn[i, :D])
        bda = jnp.kron(eye8, Wn[i, D:])
        b_r = jnp.tile(bn[i], 8)[None, :]
        if i == 0:
            x_r = _node_call_first(x_r, agg_r, bdx, bda, b_r, x_r)
            x_res = x_r
        else:
            x_r = _node_call_res(x_r, agg_r, bdx, bda, b_r, x_res)
        e_r = e_new.reshape(E_PAD // 8, 128)
        e_outs.append(e_new[:E])

    x_out = x_r.reshape(N_TAB, D)[:N]
    return x_out, e_outs[-1], jnp.stack(e_outs, axis=0)
